# baseline (speedup 1.0000x reference)
import functools

import jax
import jax.numpy as jnp
from jax import lax
from jax.experimental import pallas as pl
from jax.experimental.pallas import tpu as pltpu
from jax.experimental.pallas import tpu_sc as plsc

_SC_CORES = 2
_SC_SUBCORES = 16
_NW = _SC_CORES * _SC_SUBCORES


def _router_body(moe_ref, x_ref, rw_ref, rb_ref, nw_ref, nb_ref, eps_ref,
                 idx_ref):
    xb = x_ref[0]
    logits = jnp.dot(xb, rw_ref[0], preferred_element_type=jnp.float32)
    logits = logits + rb_ref[0]
    nlog = jnp.dot(xb, nw_ref[0], preferred_element_type=jnp.float32)
    nlog = nlog + nb_ref[0]
    sp = jnp.maximum(nlog, 0.0) + jnp.log1p(jnp.exp(-jnp.abs(nlog)))
    noisy = logits + eps_ref[0] * sp
    s, e = noisy.shape
    mx = jnp.max(noisy, axis=-1, keepdims=True)
    col = lax.broadcasted_iota(jnp.int32, (s, e), 1)
    idx = jnp.min(jnp.where(noisy == mx, col, e), axis=-1, keepdims=True)
    idx_ref[0] = idx.astype(jnp.int32)


def _router(x, router_w, router_b, noise_w, noise_b, eps, moe_i32):
    b, s, d = x.shape
    nr, _, e = router_w.shape
    rb3 = router_b.reshape(nr, 1, e)
    nb3 = noise_b.reshape(nr, 1, e)
    grid_spec = pltpu.PrefetchScalarGridSpec(
        num_scalar_prefetch=1,
        grid=(b,),
        in_specs=[
            pl.BlockSpec((1, s, d), lambda i, moe: (i, 0, 0)),
            pl.BlockSpec((1, d, e), lambda i, moe: (moe[i], 0, 0)),
            pl.BlockSpec((1, 1, e), lambda i, moe: (moe[i], 0, 0)),
            pl.BlockSpec((1, d, e), lambda i, moe: (moe[i], 0, 0)),
            pl.BlockSpec((1, 1, e), lambda i, moe: (moe[i], 0, 0)),
            pl.BlockSpec((1, s, e), lambda i, moe: (i, 0, 0)),
        ],
        out_specs=pl.BlockSpec((1, s, 1), lambda i, moe: (i, 0, 0)),
    )
    return pl.pallas_call(
        _router_body,
        grid_spec=grid_spec,
        out_shape=jax.ShapeDtypeStruct((b, s, 1), jnp.int32),
    )(moe_i32, x, router_w, rb3, noise_w, nb3, eps)


def _meta_body_factory(n, e, nt, tile_rows):
    chunk = min(512, n)
    c_chunks = n // chunk

    def body(e_ref, dest_ref, g_ref, m_ref, lo_ref, hi_ref):
        f32 = jnp.float32
        i32 = jnp.int32
        t = chunk
        iota_e_row = lax.broadcasted_iota(i32, (1, e), 1)

        def oh_chunk(c):
            e_c = e_ref[pl.ds(c * t, t), :]
            return e_c == iota_e_row

        counts_row = jnp.zeros((1, e), f32)
        for c in range(c_chunks):
            counts_row = counts_row + jnp.sum(
                oh_chunk(c).astype(f32), axis=0, keepdims=True)

        m_le = (lax.broadcasted_iota(i32, (e, e), 0)
                <= lax.broadcasted_iota(i32, (e, e), 1)).astype(f32)
        o_hi_row = lax.dot_general(
            counts_row, m_le, (((1,), (0,)), ((), ())),
            preferred_element_type=f32)
        o_lo_row = o_hi_row - counts_row

        counts_i = counts_row.astype(i32)
        o_lo_i = o_lo_row.astype(i32)
        o_hi_i = o_hi_row.astype(i32)
        shift = tile_rows.bit_length() - 1
        bm_start = jnp.right_shift(o_lo_i, shift)
        bm_end = jnp.right_shift(o_hi_i + (tile_rows - 1), shift)
        ntiles = jnp.where(counts_i > 0, bm_end - bm_start, 0)
        cum_t_row = lax.dot_general(
            ntiles.astype(f32), m_le, (((1,), (0,)), ((), ())),
            preferred_element_type=f32).astype(i32)
        start_row = cum_t_row - ntiles
        tt = jnp.max(cum_t_row)

        j_col = lax.broadcasted_iota(i32, (nt, 1), 0)
        g_col = jnp.sum((cum_t_row <= j_col).astype(i32), axis=1,
                        keepdims=True)
        g_col = jnp.minimum(g_col, e - 1)
        ohg = g_col == iota_e_row

        def at_g(row_i32):
            return jnp.sum(jnp.where(ohg, row_i32, 0), axis=1, keepdims=True)

        k_col = j_col - at_g(start_row)
        m_col = at_g(bm_start) + k_col
        lo_col = jnp.maximum(at_g(o_lo_i), m_col * tile_rows)
        hi_col = jnp.minimum(at_g(o_hi_i), (m_col + 1) * tile_rows)

        valid = j_col < tt
        g_last = jnp.max(jnp.where(valid, g_col, -1))
        m_last = jnp.max(jnp.where(valid, m_col, -1))
        g_ref[...] = jnp.where(valid, g_col, g_last)
        m_ref[...] = jnp.where(valid, m_col, m_last)
        lo_ref[...] = jnp.where(valid, lo_col, 0)
        hi_ref[...] = jnp.where(valid, hi_col, 0)

        tril = (lax.broadcasted_iota(i32, (t, t), 0)
                > lax.broadcasted_iota(i32, (t, t), 1)).astype(f32)
        carry_row = jnp.zeros((1, e), f32)
        for c in range(c_chunks):
            ohb = oh_chunk(c)
            oh_f = ohb.astype(f32)
            cum_c = lax.dot_general(tril, oh_f, (((1,), (0,)), ((), ())),
                                    preferred_element_type=f32) + carry_row
            rank_c = jnp.sum(cum_c * oh_f, axis=1, keepdims=True)
            base_c = jnp.sum(jnp.where(ohb, o_lo_row, 0.0), axis=1,
                             keepdims=True)
            dest_ref[pl.ds(c * t, t), :] = (rank_c + base_c).astype(i32)
            carry_row = carry_row + jnp.sum(oh_f, axis=0, keepdims=True)

    return body


def _make_metadata(e_col, n_tokens, n_experts, tile_rows):
    nt = n_tokens // tile_rows + n_experts - 1
    col = jax.ShapeDtypeStruct((nt, 1), jnp.int32)
    dest, g, m, lo, hi = pl.pallas_call(
        _meta_body_factory(n_tokens, n_experts, nt, tile_rows),
        grid=(1,),
        in_specs=[pl.BlockSpec((n_tokens, 1), lambda i: (0, 0))],
        out_specs=[
            pl.BlockSpec((n_tokens, 1), lambda i: (0, 0)),
            pl.BlockSpec((nt, 1), lambda i: (0, 0)),
            pl.BlockSpec((nt, 1), lambda i: (0, 0)),
            pl.BlockSpec((nt, 1), lambda i: (0, 0)),
            pl.BlockSpec((nt, 1), lambda i: (0, 0)),
        ],
        out_shape=[jax.ShapeDtypeStruct((n_tokens, 1), jnp.int32),
                   col, col, col, col],
    )(e_col)
    return (dest.reshape(n_tokens), g.reshape(nt), m.reshape(nt),
            lo.reshape(nt), hi.reshape(nt), nt)


def _sc_permute(rows_in, idx, invert):
    n, d = rows_in.shape
    per_w = n // _NW
    mesh = plsc.VectorSubcoreMesh(
        core_axis_name="c", subcore_axis_name="s",
        num_cores=_SC_CORES, num_subcores=_SC_SUBCORES)

    @functools.partial(
        pl.kernel,
        out_type=jax.ShapeDtypeStruct((n, d), rows_in.dtype),
        mesh=mesh,
        scratch_types=[
            pltpu.VMEM((per_w,), jnp.int32),
            pltpu.VMEM((per_w, d), rows_in.dtype),
            pltpu.SemaphoreType.DMA,
        ],
    )
    def _k(rows_hbm, idx_hbm, out_hbm, idx_v, rows_v, sem):
        wid = lax.axis_index("s") * _SC_CORES + lax.axis_index("c")
        base = wid * per_w
        pltpu.sync_copy(idx_hbm.at[pl.ds(base, per_w)], idx_v)
        if invert:
            pltpu.sync_copy(rows_hbm.at[pl.ds(base, per_w)], rows_v)
            pltpu.async_copy(rows_v, out_hbm.at[idx_v], sem).wait()
        else:
            pltpu.async_copy(rows_hbm.at[idx_v], rows_v, sem).wait()
            pltpu.sync_copy(rows_v, out_hbm.at[pl.ds(base, per_w)])

    return _k(rows_in, idx)


def _gmm_body(g_ref, m_ref, lo_ref, hi_ref, xs_ref, w1_ref, b1_ref, w2_ref,
              b2_ref, out_ref):
    t = pl.program_id(0)
    tile_rows = xs_ref.shape[0]
    first = jnp.logical_or(t == 0, m_ref[t] != m_ref[jnp.maximum(t - 1, 0)])
    active = hi_ref[t] > lo_ref[t]

    @pl.when(jnp.logical_and(first, jnp.logical_not(active)))
    def _():
        out_ref[...] = jnp.zeros_like(out_ref)

    @pl.when(active)
    def _():
        xs = xs_ref[...]
        h = jnp.dot(xs, w1_ref[0], preferred_element_type=jnp.float32)
        h = jnp.maximum(h + b1_ref[0], 0.0)
        out = jnp.dot(h, w2_ref[0], preferred_element_type=jnp.float32)
        out = out + b2_ref[0]
        row0 = m_ref[t] * tile_rows
        rows = row0 + lax.broadcasted_iota(jnp.int32, (tile_rows, 1), 0)
        mask = jnp.logical_and(rows >= lo_ref[t], rows < hi_ref[t])
        contrib = jnp.where(mask, out, 0.0)

        @pl.when(first)
        def _():
            out_ref[...] = contrib

        @pl.when(jnp.logical_not(first))
        def _():
            out_ref[...] = out_ref[...] + contrib


def _gmm(x_sorted, w1, b1, w2, b2, g, m, lo, hi, nt, tile_rows):
    n, d = x_sorted.shape
    e, _, dff = w1.shape
    b1r = b1.reshape(e, 1, dff)
    b2r = b2.reshape(e, 1, d)
    grid_spec = pltpu.PrefetchScalarGridSpec(
        num_scalar_prefetch=4,
        grid=(nt,),
        in_specs=[
            pl.BlockSpec((tile_rows, d), lambda t, g, m, lo, hi: (m[t], 0)),
            pl.BlockSpec((1, d, dff), lambda t, g, m, lo, hi: (g[t], 0, 0)),
            pl.BlockSpec((1, 1, dff), lambda t, g, m, lo, hi: (g[t], 0, 0)),
            pl.BlockSpec((1, dff, d), lambda t, g, m, lo, hi: (g[t], 0, 0)),
            pl.BlockSpec((1, 1, d), lambda t, g, m, lo, hi: (g[t], 0, 0)),
        ],
        out_specs=pl.BlockSpec((tile_rows, d),
                               lambda t, g, m, lo, hi: (m[t], 0)),
    )
    return pl.pallas_call(
        _gmm_body,
        grid_spec=grid_spec,
        out_shape=jax.ShapeDtypeStruct((n, d), jnp.float32),
        compiler_params=pltpu.CompilerParams(
            dimension_semantics=("arbitrary",)),
    )(g, m, lo, hi, x_sorted, w1, b1r, w2, b2r)


def kernel(x, router_w, router_b, noise_w, noise_b, w1, b1, w2, b2, moe):
    b, s, d = x.shape
    e = w1.shape[0]
    n = b * s
    tile_rows = 64

    with jax.ensure_compile_time_eval():
        noise_key = jax.random.key(42)
        eps = jnp.stack([
            jax.random.normal(jax.random.fold_in(noise_key, i), (s, e),
                              dtype=jnp.float32)
            for i in range(b)
        ])

    moe_i32 = moe.astype(jnp.int32)
    indices = _router(x, router_w, router_b, noise_w, noise_b, eps, moe_i32)

    e_col = indices.reshape(n, 1)
    dest, g, m, lo, hi, nt = _make_metadata(e_col, n, e, tile_rows)

    x_flat = x.reshape(n, d)
    x_sorted = _sc_permute(x_flat, dest, invert=True)
    out_sorted = _gmm(x_sorted, w1, b1, w2, b2, g, m, lo, hi, nt, tile_rows)
    final_flat = _sc_permute(out_sorted, dest, invert=False)

    return final_flat.reshape(b, s, d), indices


# speedup vs baseline: 1.1844x; 1.1844x over previous
import functools

import jax
import jax.numpy as jnp
from jax import lax
from jax.experimental import pallas as pl
from jax.experimental.pallas import tpu as pltpu
from jax.experimental.pallas import tpu_sc as plsc

_SC_CORES = 2
_SC_SUBCORES = 16
_NW = _SC_CORES * _SC_SUBCORES


def _router_body(moe_ref, x_ref, rw_ref, rb_ref, nw_ref, nb_ref, eps_ref,
                 idx_ref):
    xb = x_ref[0]
    logits = jnp.dot(xb, rw_ref[0], preferred_element_type=jnp.float32)
    logits = logits + rb_ref[0]
    nlog = jnp.dot(xb, nw_ref[0], preferred_element_type=jnp.float32)
    nlog = nlog + nb_ref[0]
    sp = jnp.maximum(nlog, 0.0) + jnp.log1p(jnp.exp(-jnp.abs(nlog)))
    noisy = logits + eps_ref[0] * sp
    s, e = noisy.shape
    mx = jnp.max(noisy, axis=-1, keepdims=True)
    col = lax.broadcasted_iota(jnp.int32, (s, e), 1)
    idx = jnp.min(jnp.where(noisy == mx, col, e), axis=-1, keepdims=True)
    idx_ref[0] = idx.astype(jnp.int32)


def _router(x, router_w, router_b, noise_w, noise_b, eps, moe_i32):
    b, s, d = x.shape
    nr, _, e = router_w.shape
    rb3 = router_b.reshape(nr, 1, e)
    nb3 = noise_b.reshape(nr, 1, e)
    grid_spec = pltpu.PrefetchScalarGridSpec(
        num_scalar_prefetch=1,
        grid=(b,),
        in_specs=[
            pl.BlockSpec((1, s, d), lambda i, moe: (i, 0, 0)),
            pl.BlockSpec((1, d, e), lambda i, moe: (moe[i], 0, 0)),
            pl.BlockSpec((1, 1, e), lambda i, moe: (moe[i], 0, 0)),
            pl.BlockSpec((1, d, e), lambda i, moe: (moe[i], 0, 0)),
            pl.BlockSpec((1, 1, e), lambda i, moe: (moe[i], 0, 0)),
            pl.BlockSpec((1, s, e), lambda i, moe: (i, 0, 0)),
        ],
        out_specs=pl.BlockSpec((1, s, 1), lambda i, moe: (i, 0, 0)),
    )
    return pl.pallas_call(
        _router_body,
        grid_spec=grid_spec,
        out_shape=jax.ShapeDtypeStruct((b, s, 1), jnp.int32),
    )(moe_i32, x, router_w, rb3, noise_w, nb3, eps)


def _meta_body_factory(n, e, nt, tile_rows):
    chunk = min(512, n)
    c_chunks = n // chunk

    def body(e_ref, dest_ref, g_ref, m_ref, lo_ref, hi_ref):
        f32 = jnp.float32
        i32 = jnp.int32
        t = chunk
        iota_e_row = lax.broadcasted_iota(i32, (1, e), 1)

        def oh_chunk(c):
            e_c = e_ref[pl.ds(c * t, t), :]
            return e_c == iota_e_row

        counts_row = jnp.zeros((1, e), f32)
        for c in range(c_chunks):
            counts_row = counts_row + jnp.sum(
                oh_chunk(c).astype(f32), axis=0, keepdims=True)

        m_le = (lax.broadcasted_iota(i32, (e, e), 0)
                <= lax.broadcasted_iota(i32, (e, e), 1)).astype(f32)
        o_hi_row = lax.dot_general(
            counts_row, m_le, (((1,), (0,)), ((), ())),
            preferred_element_type=f32)
        o_lo_row = o_hi_row - counts_row

        counts_i = counts_row.astype(i32)
        o_lo_i = o_lo_row.astype(i32)
        o_hi_i = o_hi_row.astype(i32)
        shift = tile_rows.bit_length() - 1
        bm_start = jnp.right_shift(o_lo_i, shift)
        bm_end = jnp.right_shift(o_hi_i + (tile_rows - 1), shift)
        ntiles = jnp.where(counts_i > 0, bm_end - bm_start, 0)
        cum_t_row = lax.dot_general(
            ntiles.astype(f32), m_le, (((1,), (0,)), ((), ())),
            preferred_element_type=f32).astype(i32)
        start_row = cum_t_row - ntiles
        tt = jnp.max(cum_t_row)

        j_col = lax.broadcasted_iota(i32, (nt, 1), 0)
        g_col = jnp.sum((cum_t_row <= j_col).astype(i32), axis=1,
                        keepdims=True)
        g_col = jnp.minimum(g_col, e - 1)
        ohg = g_col == iota_e_row

        def at_g(row_i32):
            return jnp.sum(jnp.where(ohg, row_i32, 0), axis=1, keepdims=True)

        k_col = j_col - at_g(start_row)
        m_col = at_g(bm_start) + k_col
        lo_col = jnp.maximum(at_g(o_lo_i), m_col * tile_rows)
        hi_col = jnp.minimum(at_g(o_hi_i), (m_col + 1) * tile_rows)

        valid = j_col < tt
        g_last = jnp.max(jnp.where(valid, g_col, -1))
        m_last = jnp.max(jnp.where(valid, m_col, -1))
        g_ref[...] = jnp.where(valid, g_col, g_last)
        m_ref[...] = jnp.where(valid, m_col, m_last)
        lo_ref[...] = jnp.where(valid, lo_col, 0)
        hi_ref[...] = jnp.where(valid, hi_col, 0)

        tril = (lax.broadcasted_iota(i32, (t, t), 0)
                > lax.broadcasted_iota(i32, (t, t), 1)).astype(f32)
        carry_row = jnp.zeros((1, e), f32)
        for c in range(c_chunks):
            ohb = oh_chunk(c)
            oh_f = ohb.astype(f32)
            cum_c = lax.dot_general(tril, oh_f, (((1,), (0,)), ((), ())),
                                    preferred_element_type=f32) + carry_row
            rank_c = jnp.sum(cum_c * oh_f, axis=1, keepdims=True)
            base_c = jnp.sum(jnp.where(ohb, o_lo_row, 0.0), axis=1,
                             keepdims=True)
            dest_ref[pl.ds(c * t, t), :] = (rank_c + base_c).astype(i32)
            carry_row = carry_row + jnp.sum(oh_f, axis=0, keepdims=True)

    return body


def _make_metadata(e_col, n_tokens, n_experts, tile_rows):
    nt = n_tokens // tile_rows + n_experts - 1
    col = jax.ShapeDtypeStruct((nt, 1), jnp.int32)
    dest, g, m, lo, hi = pl.pallas_call(
        _meta_body_factory(n_tokens, n_experts, nt, tile_rows),
        grid=(1,),
        in_specs=[pl.BlockSpec((n_tokens, 1), lambda i: (0, 0))],
        out_specs=[
            pl.BlockSpec((n_tokens, 1), lambda i: (0, 0)),
            pl.BlockSpec((nt, 1), lambda i: (0, 0)),
            pl.BlockSpec((nt, 1), lambda i: (0, 0)),
            pl.BlockSpec((nt, 1), lambda i: (0, 0)),
            pl.BlockSpec((nt, 1), lambda i: (0, 0)),
        ],
        out_shape=[jax.ShapeDtypeStruct((n_tokens, 1), jnp.int32),
                   col, col, col, col],
    )(e_col)
    return (dest.reshape(n_tokens), g.reshape(nt), m.reshape(nt),
            lo.reshape(nt), hi.reshape(nt), nt)


def _sc_permute(rows_in, idx, invert):
    n, d = rows_in.shape
    per_w = n // _NW
    mesh = plsc.VectorSubcoreMesh(
        core_axis_name="c", subcore_axis_name="s",
        num_cores=_SC_CORES, num_subcores=_SC_SUBCORES)

    @functools.partial(
        pl.kernel,
        out_type=jax.ShapeDtypeStruct((n, d), rows_in.dtype),
        mesh=mesh,
        scratch_types=[
            pltpu.VMEM((per_w,), jnp.int32),
            pltpu.VMEM((per_w, d), rows_in.dtype),
            pltpu.SemaphoreType.DMA,
        ],
    )
    def _k(rows_hbm, idx_hbm, out_hbm, idx_v, rows_v, sem):
        wid = lax.axis_index("s") * _SC_CORES + lax.axis_index("c")
        base = wid * per_w
        pltpu.sync_copy(idx_hbm.at[pl.ds(base, per_w)], idx_v)
        if invert:
            pltpu.sync_copy(rows_hbm.at[pl.ds(base, per_w)], rows_v)
            pltpu.async_copy(rows_v, out_hbm.at[idx_v], sem).wait()
        else:
            pltpu.async_copy(rows_hbm.at[idx_v], rows_v, sem).wait()
            pltpu.sync_copy(rows_v, out_hbm.at[pl.ds(base, per_w)])

    return _k(rows_in, idx)


def _gmm_body(g_ref, m_ref, lo_ref, hi_ref, xs_ref, w1_ref, b1_ref, w2_ref,
              b2_ref, out_ref):
    t = pl.program_id(0)
    tile_rows = xs_ref.shape[0]
    first = jnp.logical_or(t == 0, m_ref[t] != m_ref[jnp.maximum(t - 1, 0)])
    active = hi_ref[t] > lo_ref[t]

    @pl.when(jnp.logical_and(first, jnp.logical_not(active)))
    def _():
        out_ref[...] = jnp.zeros_like(out_ref)

    @pl.when(active)
    def _():
        xs = xs_ref[...]
        h = jnp.dot(xs, w1_ref[0], preferred_element_type=jnp.float32)
        h = jnp.maximum(h + b1_ref[0], 0.0)
        out = jnp.dot(h, w2_ref[0], preferred_element_type=jnp.float32)
        out = out + b2_ref[0]
        row0 = m_ref[t] * tile_rows
        rows = row0 + lax.broadcasted_iota(jnp.int32, (tile_rows, 1), 0)
        mask = jnp.logical_and(rows >= lo_ref[t], rows < hi_ref[t])
        contrib = jnp.where(mask, out, 0.0)

        @pl.when(first)
        def _():
            out_ref[...] = contrib

        @pl.when(jnp.logical_not(first))
        def _():
            out_ref[...] = out_ref[...] + contrib


def _gmm(x_sorted, w1, b1, w2, b2, g, m, lo, hi, nt, tile_rows):
    n, d = x_sorted.shape
    e, _, dff = w1.shape
    b1r = b1.reshape(e, 1, dff)
    b2r = b2.reshape(e, 1, d)
    grid_spec = pltpu.PrefetchScalarGridSpec(
        num_scalar_prefetch=4,
        grid=(nt,),
        in_specs=[
            pl.BlockSpec((tile_rows, d), lambda t, g, m, lo, hi: (m[t], 0)),
            pl.BlockSpec((1, d, dff), lambda t, g, m, lo, hi: (g[t], 0, 0)),
            pl.BlockSpec((1, 1, dff), lambda t, g, m, lo, hi: (g[t], 0, 0)),
            pl.BlockSpec((1, dff, d), lambda t, g, m, lo, hi: (g[t], 0, 0)),
            pl.BlockSpec((1, 1, d), lambda t, g, m, lo, hi: (g[t], 0, 0)),
        ],
        out_specs=pl.BlockSpec((tile_rows, d),
                               lambda t, g, m, lo, hi: (m[t], 0)),
    )
    return pl.pallas_call(
        _gmm_body,
        grid_spec=grid_spec,
        out_shape=jax.ShapeDtypeStruct((n, d), jnp.float32),
        compiler_params=pltpu.CompilerParams(
            dimension_semantics=("arbitrary",)),
    )(g, m, lo, hi, x_sorted, w1, b1r, w2, b2r)


def kernel(x, router_w, router_b, noise_w, noise_b, w1, b1, w2, b2, moe):
    b, s, d = x.shape
    e = w1.shape[0]
    n = b * s
    tile_rows = 128 if n <= 256 else 256

    with jax.ensure_compile_time_eval():
        noise_key = jax.random.key(42)
        eps = jnp.stack([
            jax.random.normal(jax.random.fold_in(noise_key, i), (s, e),
                              dtype=jnp.float32)
            for i in range(b)
        ])

    moe_i32 = moe.astype(jnp.int32)
    indices = _router(x, router_w, router_b, noise_w, noise_b, eps, moe_i32)

    e_col = indices.reshape(n, 1)
    dest, g, m, lo, hi, nt = _make_metadata(e_col, n, e, tile_rows)

    x_flat = x.reshape(n, d)
    x_sorted = _sc_permute(x_flat, dest, invert=True)
    out_sorted = _gmm(x_sorted, w1, b1, w2, b2, g, m, lo, hi, nt, tile_rows)
    final_flat = _sc_permute(out_sorted, dest, invert=False)

    return final_flat.reshape(b, s, d), indices
